# baseline (speedup 1.0000x reference)
import functools

import jax
import jax.numpy as jnp
from jax import lax
from jax.experimental import pallas as pl
from jax.experimental.pallas import tpu as pltpu
from jax.experimental.pallas import tpu_sc as plsc

N = 10000
E = 320000
F_IN = 128
HID = 32
F_OUT = 16
NC, NS, L = 2, 16, 16
NW = NC * NS
CH = 128
NCH = E // CH
KB = 80
KT = NCH - KB * (NW - 1)
KTU = -(-KT // 8) * 8
N_ACC = 10240
RPT = N_ACC // NS
ZR = 40
NB = 6
GRID = 8
BR = N_ACC // GRID

_MESH = plsc.VectorSubcoreMesh(core_axis_name="c", subcore_axis_name="s")


@functools.partial(
    pl.kernel,
    out_type=jax.ShapeDtypeStruct((NC, N_ACC), jnp.float32),
    mesh=_MESH,
    scratch_types=[
        pltpu.VMEM((KB, CH), jnp.int32),
        pltpu.VMEM((CH,), jnp.float32),
        pltpu.VMEM((RPT,), jnp.float32),
        pltpu.VMEM_SHARED((N_ACC,), jnp.float32),
        pltpu.SemaphoreType.DMA,
    ],
)
def _deg_kernel(dstr, out, dst_v, ones_v, zbuf, acc, ssem):
    c = lax.axis_index("c")
    s = lax.axis_index("s")
    wid = s * NC + c
    start = KB * wid
    kw = jnp.where(wid < NW - 1, KB, KT)

    def _fill(i, _):
        ones_v[pl.ds(i * L, L)] = jnp.full((L,), 1.0, jnp.float32)
        return 0

    lax.fori_loop(0, CH // L, _fill, 0)

    def _zb(i, _):
        zbuf[pl.ds(i * L, L)] = jnp.zeros((L,), jnp.float32)
        return 0

    lax.fori_loop(0, RPT // L, _zb, 0)
    pltpu.sync_copy(zbuf, acc.at[pl.ds(s * RPT, RPT)])
    @pl.when(wid < NW - 1)
    def _():
        pltpu.sync_copy(dstr.at[pl.ds(start, KB)], dst_v)

    @pl.when(wid == NW - 1)
    def _():
        pltpu.sync_copy(dstr.at[pl.ds(start, KTU)], dst_v.at[pl.ds(0, KTU)])

    plsc.subcore_barrier()

    def _scat(k, _):
        @pl.when(k >= 4)
        def _():
            pltpu.make_async_copy(ones_v, acc.at[dst_v.at[k - 4]], ssem).wait()

        pltpu.async_copy(ones_v, acc.at[dst_v.at[k]], ssem, add=True)
        return 0

    lax.fori_loop(0, kw, _scat, 0)

    def _drain(k, _):
        pltpu.make_async_copy(ones_v, acc.at[dst_v.at[k]], ssem).wait()
        return 0

    lax.fori_loop(kw - 4, kw, _drain, 0)
    plsc.subcore_barrier()
    pltpu.sync_copy(acc.at[pl.ds(s * RPT, RPT)], out.at[c, pl.ds(s * RPT, RPT)])


@functools.partial(
    pl.kernel,
    out_type=jax.ShapeDtypeStruct((NC, N_ACC, HID), jnp.float32),
    mesh=_MESH,
    scratch_types=[
        pltpu.VMEM((KB, CH), jnp.int32),
        pltpu.VMEM((KB, CH), jnp.int32),
        pltpu.VMEM((NB, CH, HID), jnp.float32),
        pltpu.VMEM((ZR, HID), jnp.float32),
        pltpu.VMEM_SHARED((N_ACC, HID), jnp.float32),
        pltpu.SemaphoreType.DMA,
        pltpu.SemaphoreType.DMA,
    ],
    compiler_params=pltpu.CompilerParams(use_tc_tiling_on_sc=False),
)
def _spmm_kernel(table, srcr, dstr, out, src_v, dst_v, rowbuf, zbuf, acc, gsem, ssem):
    c = lax.axis_index("c")
    s = lax.axis_index("s")
    wid = s * NC + c
    start = KB * wid
    kw = jnp.where(wid < NW - 1, KB, KT)

    def _zb(i, _):
        zbuf[i // 2, pl.ds((i % 2) * L, L)] = jnp.zeros((L,), jnp.float32)
        return 0

    lax.fori_loop(0, ZR * (HID // L), _zb, 0)

    def _zc(j, _):
        pltpu.sync_copy(zbuf, acc.at[pl.ds(s * RPT + j * ZR, ZR)])
        return 0

    lax.fori_loop(0, RPT // ZR, _zc, 0)
    @pl.when(wid < NW - 1)
    def _():
        pltpu.sync_copy(srcr.at[pl.ds(start, KB)], src_v)
        pltpu.sync_copy(dstr.at[pl.ds(start, KB)], dst_v)

    @pl.when(wid == NW - 1)
    def _():
        pltpu.sync_copy(srcr.at[pl.ds(start, KTU)], src_v.at[pl.ds(0, KTU)])
        pltpu.sync_copy(dstr.at[pl.ds(start, KTU)], dst_v.at[pl.ds(0, KTU)])

    plsc.subcore_barrier()

    for j in range(NB - 2):
        pltpu.async_copy(table.at[src_v.at[j]], rowbuf.at[j], gsem)

    def _mb(k, _):
        b = lax.rem(k, NB)
        pb = lax.rem(k + NB - 2, NB)

        @pl.when(k >= 2)
        def _():
            pltpu.make_async_copy(rowbuf.at[pb], acc.at[dst_v.at[k - 2]], ssem).wait()

        @pl.when(k + NB - 2 < kw)
        def _():
            pltpu.async_copy(table.at[src_v.at[k + NB - 2]], rowbuf.at[pb], gsem)

        pltpu.make_async_copy(table.at[src_v.at[k]], rowbuf.at[b], gsem).wait()
        pltpu.async_copy(rowbuf.at[b], acc.at[dst_v.at[k]], ssem, add=True)
        return 0

    lax.fori_loop(0, kw, _mb, 0)
    pltpu.make_async_copy(rowbuf.at[lax.rem(kw - 2, NB)], acc.at[dst_v.at[kw - 2]], ssem).wait()
    pltpu.make_async_copy(rowbuf.at[lax.rem(kw - 1, NB)], acc.at[dst_v.at[kw - 1]], ssem).wait()
    plsc.subcore_barrier()
    pltpu.sync_copy(acc.at[pl.ds(s * RPT, RPT)], out.at[c, pl.ds(s * RPT, RPT)])


def _enc_body(degp_ref, x_ref, w1_ref, pt_ref, dinv_ref):
    deg = degp_ref[0, :] + degp_ref[1, :] + 1.0
    dinv = lax.rsqrt(jnp.maximum(deg, 1.0))[:, None]
    p = jnp.dot(x_ref[...], w1_ref[...], preferred_element_type=jnp.float32)
    pt_ref[pl.ds(0, N)] = p * dinv[:N]
    pt_ref[pl.ds(N, N_ACC - N)] = jnp.zeros((N_ACC - N, HID), jnp.float32)
    dinv_ref[...] = dinv


_enc = pl.pallas_call(
    _enc_body,
    out_shape=[
        jax.ShapeDtypeStruct((N_ACC, HID), jnp.float32),
        jax.ShapeDtypeStruct((N_ACC, 1), jnp.float32),
    ],
)


def _mid_body(y_ref, pt_ref, dinv_ref, b1_ref, ht_ref):
    t = y_ref[0] + y_ref[1] + pt_ref[...]
    dinv = dinv_ref[...]
    h = jnp.maximum(dinv * t + b1_ref[...], 0.0)
    ht_ref[...] = h * dinv


_mid = pl.pallas_call(
    _mid_body,
    grid=(GRID,),
    in_specs=[
        pl.BlockSpec((2, BR, HID), lambda i: (0, i, 0)),
        pl.BlockSpec((BR, HID), lambda i: (i, 0)),
        pl.BlockSpec((BR, 1), lambda i: (i, 0)),
        pl.BlockSpec((1, HID), lambda i: (0, 0)),
    ],
    out_specs=pl.BlockSpec((BR, HID), lambda i: (i, 0)),
    out_shape=jax.ShapeDtypeStruct((N_ACC, HID), jnp.float32),
)


def _dec_body(z_ref, ht_ref, dinv_ref, wmu_ref, bmu_ref, wls_ref, bls_ref,
              eps_ref, zout_ref):
    agg = dinv_ref[...] * (z_ref[0] + z_ref[1] + ht_ref[...])
    mu = jnp.dot(agg, wmu_ref[...], preferred_element_type=jnp.float32) + bmu_ref[...]
    ls = jnp.dot(agg, wls_ref[...], preferred_element_type=jnp.float32) + bls_ref[...]
    zout_ref[...] = mu + eps_ref[...] * jnp.exp(ls)


_dec = pl.pallas_call(
    _dec_body,
    grid=(GRID,),
    in_specs=[
        pl.BlockSpec((2, BR, HID), lambda i: (0, i, 0)),
        pl.BlockSpec((BR, HID), lambda i: (i, 0)),
        pl.BlockSpec((BR, 1), lambda i: (i, 0)),
        pl.BlockSpec((HID, F_OUT), lambda i: (0, 0)),
        pl.BlockSpec((1, F_OUT), lambda i: (0, 0)),
        pl.BlockSpec((HID, F_OUT), lambda i: (0, 0)),
        pl.BlockSpec((1, F_OUT), lambda i: (0, 0)),
        pl.BlockSpec((BR, F_OUT), lambda i: (i, 0)),
    ],
    out_specs=pl.BlockSpec((BR, F_OUT), lambda i: (i, 0)),
    out_shape=jax.ShapeDtypeStruct((N, F_OUT), jnp.float32),
)


def kernel(x, edge_index, W1, b1, Wmu, bmu, Wls, bls, eps):
    srcr = edge_index[0].reshape(NCH, CH)
    dstr = edge_index[1].reshape(NCH, CH)

    degp = _deg_kernel(dstr)
    pt, dinv = _enc(degp, x, W1)
    y = _spmm_kernel(pt, srcr, dstr)
    ht = _mid(y, pt, dinv, b1.reshape(1, HID))
    z2 = _spmm_kernel(ht, srcr, dstr)
    return _dec(z2, ht, dinv, Wmu, bmu.reshape(1, F_OUT), Wls, bls.reshape(1, F_OUT), eps)


# speedup vs baseline: 1.1274x; 1.1274x over previous
import functools

import jax
import jax.numpy as jnp
from jax import lax
from jax.experimental import pallas as pl
from jax.experimental.pallas import tpu as pltpu
from jax.experimental.pallas import tpu_sc as plsc

N = 10000
E = 320000
F_IN = 128
HID = 32
F_OUT = 16
NC, NS, L = 2, 16, 16
NW = NC * NS
CH = 128
NCH = E // CH
KB = 80
KT = NCH - KB * (NW - 1)
KTU = -(-KT // 8) * 8
N_ACC = 10240
RPT = N_ACC // NS
ZR = 40
NB = 6
NR4 = N_ACC // 4
XR4 = N // 4
NR8 = N_ACC // 8
MGRID, MBR = 8, N_ACC // 4 // 8
DGRID, DBR = 5, N_ACC // 8 // 5

_MESH = plsc.VectorSubcoreMesh(core_axis_name="c", subcore_axis_name="s")


@functools.partial(
    pl.kernel,
    out_type=jax.ShapeDtypeStruct((NC, N_ACC, HID), jnp.float32),
    mesh=_MESH,
    scratch_types=[
        pltpu.VMEM((KB, CH), jnp.int32),
        pltpu.VMEM((CH, HID), jnp.float32),
        pltpu.VMEM((ZR, HID), jnp.float32),
        pltpu.VMEM_SHARED((N_ACC, HID), jnp.float32),
        pltpu.SemaphoreType.DMA,
    ],
    compiler_params=pltpu.CompilerParams(use_tc_tiling_on_sc=False),
)
def _deg_kernel(dstr, out, dst_v, ones_m, zbuf, acc, ssem):
    c = lax.axis_index("c")
    s = lax.axis_index("s")
    wid = s * NC + c
    start = KB * wid
    kw = jnp.where(wid < NW - 1, KB, KT)

    def _fill(i, _):
        ones_m[i // 2, pl.ds((i % 2) * L, L)] = jnp.full((L,), 1.0, jnp.float32)
        return 0

    lax.fori_loop(0, CH * (HID // L), _fill, 0)

    def _zb(i, _):
        zbuf[i // 2, pl.ds((i % 2) * L, L)] = jnp.zeros((L,), jnp.float32)
        return 0

    lax.fori_loop(0, ZR * (HID // L), _zb, 0)

    def _zc(j, _):
        pltpu.sync_copy(zbuf, acc.at[pl.ds(s * RPT + j * ZR, ZR)])
        return 0

    lax.fori_loop(0, RPT // ZR, _zc, 0)

    @pl.when(wid < NW - 1)
    def _():
        pltpu.sync_copy(dstr.at[pl.ds(start, KB)], dst_v)

    @pl.when(wid == NW - 1)
    def _():
        pltpu.sync_copy(dstr.at[pl.ds(start, KTU)], dst_v.at[pl.ds(0, KTU)])

    plsc.subcore_barrier()

    def _scat(k, _):
        @pl.when(k >= 4)
        def _():
            pltpu.make_async_copy(ones_m, acc.at[dst_v.at[k - 4]], ssem).wait()

        pltpu.async_copy(ones_m, acc.at[dst_v.at[k]], ssem, add=True)
        return 0

    lax.fori_loop(0, kw, _scat, 0)

    def _drain(k, _):
        pltpu.make_async_copy(ones_m, acc.at[dst_v.at[k]], ssem).wait()
        return 0

    lax.fori_loop(kw - 4, kw, _drain, 0)
    plsc.subcore_barrier()
    pltpu.sync_copy(acc.at[pl.ds(s * RPT, RPT)], out.at[c, pl.ds(s * RPT, RPT)])


@functools.partial(
    pl.kernel,
    out_type=jax.ShapeDtypeStruct((NC, N_ACC, HID), jnp.float32),
    mesh=_MESH,
    scratch_types=[
        pltpu.VMEM((KB, CH), jnp.int32),
        pltpu.VMEM((KB, CH), jnp.int32),
        pltpu.VMEM((NB, CH, HID), jnp.float32),
        pltpu.VMEM((ZR, HID), jnp.float32),
        pltpu.VMEM_SHARED((N_ACC, HID), jnp.float32),
        pltpu.SemaphoreType.DMA,
        pltpu.SemaphoreType.DMA,
    ],
    compiler_params=pltpu.CompilerParams(use_tc_tiling_on_sc=False),
)
def _spmm_kernel(table, srcr, dstr, out, src_v, dst_v, rowbuf, zbuf, acc, gsem, ssem):
    c = lax.axis_index("c")
    s = lax.axis_index("s")
    wid = s * NC + c
    start = KB * wid
    kw = jnp.where(wid < NW - 1, KB, KT)

    def _zb(i, _):
        zbuf[i // 2, pl.ds((i % 2) * L, L)] = jnp.zeros((L,), jnp.float32)
        return 0

    lax.fori_loop(0, ZR * (HID // L), _zb, 0)

    def _zc(j, _):
        pltpu.sync_copy(zbuf, acc.at[pl.ds(s * RPT + j * ZR, ZR)])
        return 0

    lax.fori_loop(0, RPT // ZR, _zc, 0)
    @pl.when(wid < NW - 1)
    def _():
        pltpu.sync_copy(srcr.at[pl.ds(start, KB)], src_v)
        pltpu.sync_copy(dstr.at[pl.ds(start, KB)], dst_v)

    @pl.when(wid == NW - 1)
    def _():
        pltpu.sync_copy(srcr.at[pl.ds(start, KTU)], src_v.at[pl.ds(0, KTU)])
        pltpu.sync_copy(dstr.at[pl.ds(start, KTU)], dst_v.at[pl.ds(0, KTU)])

    plsc.subcore_barrier()

    for j in range(NB - 2):
        pltpu.async_copy(table.at[src_v.at[j]], rowbuf.at[j], gsem)

    def _mb(k, _):
        b = lax.rem(k, NB)
        pb = lax.rem(k + NB - 2, NB)

        @pl.when(k >= 2)
        def _():
            pltpu.make_async_copy(rowbuf.at[pb], acc.at[dst_v.at[k - 2]], ssem).wait()

        @pl.when(k + NB - 2 < kw)
        def _():
            pltpu.async_copy(table.at[src_v.at[k + NB - 2]], rowbuf.at[pb], gsem)

        pltpu.make_async_copy(table.at[src_v.at[k]], rowbuf.at[b], gsem).wait()
        pltpu.async_copy(rowbuf.at[b], acc.at[dst_v.at[k]], ssem, add=True)
        return 0

    lax.fori_loop(0, kw, _mb, 0)
    pltpu.make_async_copy(rowbuf.at[lax.rem(kw - 2, NB)], acc.at[dst_v.at[kw - 2]], ssem).wait()
    pltpu.make_async_copy(rowbuf.at[lax.rem(kw - 1, NB)], acc.at[dst_v.at[kw - 1]], ssem).wait()
    plsc.subcore_barrier()
    pltpu.sync_copy(acc.at[pl.ds(s * RPT, RPT)], out.at[c, pl.ds(s * RPT, RPT)])


def _enc_body(degp_ref, x_ref, w1_ref, pt_ref, dinv_ref):
    deg4 = degp_ref[0] + degp_ref[1] + 1.0
    dinv4 = lax.rsqrt(jnp.maximum(deg4, 1.0))
    p = jnp.dot(x_ref[...], w1_ref[...], preferred_element_type=jnp.float32)
    pt_ref[pl.ds(0, XR4)] = p * dinv4[:XR4]
    dinv_ref[...] = dinv4


_enc = pl.pallas_call(
    _enc_body,
    out_shape=[
        jax.ShapeDtypeStruct((NR4, 128), jnp.float32),
        jax.ShapeDtypeStruct((NR4, 128), jnp.float32),
    ],
)


def _mid_body(y_ref, pt_ref, dinv_ref, b1_ref, ht_ref):
    t = y_ref[0] + y_ref[1] + pt_ref[...]
    dinv = dinv_ref[...]
    h = jnp.maximum(dinv * t + b1_ref[...], 0.0)
    ht_ref[...] = h * dinv


_mid = pl.pallas_call(
    _mid_body,
    grid=(MGRID,),
    in_specs=[
        pl.BlockSpec((2, MBR, 128), lambda i: (0, i, 0)),
        pl.BlockSpec((MBR, 128), lambda i: (i, 0)),
        pl.BlockSpec((MBR, 128), lambda i: (i, 0)),
        pl.BlockSpec((1, 128), lambda i: (0, 0)),
    ],
    out_specs=pl.BlockSpec((MBR, 128), lambda i: (i, 0)),
    out_shape=jax.ShapeDtypeStruct((NR4, 128), jnp.float32),
)


def _dec_body(z_ref, ht_ref, dinv_ref, wmu_ref, bmu_ref, wls_ref, bls_ref,
              eps_ref, z8_ref):
    agg = dinv_ref[...] * (z_ref[0] + z_ref[1] + ht_ref[...])
    mu = jnp.dot(agg, wmu_ref[...], preferred_element_type=jnp.float32) + bmu_ref[...]
    ls = jnp.dot(agg, wls_ref[...], preferred_element_type=jnp.float32) + bls_ref[...]
    z8_ref[...] = mu + eps_ref[...] * jnp.exp(ls)


_dec = pl.pallas_call(
    _dec_body,
    grid=(DGRID,),
    in_specs=[
        pl.BlockSpec((2, DBR, 256), lambda i: (0, i, 0)),
        pl.BlockSpec((DBR, 256), lambda i: (i, 0)),
        pl.BlockSpec((DBR, 256), lambda i: (i, 0)),
        pl.BlockSpec((256, 128), lambda i: (0, 0)),
        pl.BlockSpec((1, 128), lambda i: (0, 0)),
        pl.BlockSpec((256, 128), lambda i: (0, 0)),
        pl.BlockSpec((1, 128), lambda i: (0, 0)),
        pl.BlockSpec((DBR, 128), lambda i: (i, 0)),
    ],
    out_specs=pl.BlockSpec((DBR, 128), lambda i: (i, 0)),
    out_shape=jax.ShapeDtypeStruct((NR8, 128), jnp.float32),
)


def kernel(x, edge_index, W1, b1, Wmu, bmu, Wls, bls, eps):
    srcr = edge_index[0].reshape(NCH, CH)
    dstr = edge_index[1].reshape(NCH, CH)
    x4 = x.reshape(XR4, 4 * F_IN)
    f32 = jnp.float32
    w1b = jnp.kron(jnp.eye(4, dtype=f32), W1)
    wmub = jnp.kron(jnp.eye(8, dtype=f32), Wmu)
    wlsb = jnp.kron(jnp.eye(8, dtype=f32), Wls)
    b1b = jnp.tile(b1, 4)[None, :]
    bmub = jnp.tile(bmu, 8)[None, :]
    blsb = jnp.tile(bls, 8)[None, :]
    eps8 = jnp.pad(eps.reshape(N // 8, 128), ((0, NR8 - N // 8), (0, 0)))

    degp = _deg_kernel(dstr)
    pt4, dinv4 = _enc(degp.reshape(NC, NR4, 128), x4, w1b)
    y = _spmm_kernel(pt4.reshape(N_ACC, HID), srcr, dstr)
    ht4 = _mid(y.reshape(NC, NR4, 128), pt4, dinv4, b1b)
    z2 = _spmm_kernel(ht4.reshape(N_ACC, HID), srcr, dstr)
    z8 = _dec(z2.reshape(NC, NR8, 256), ht4.reshape(NR8, 256),
              dinv4.reshape(NR8, 256), wmub, bmub, wlsb, blsb, eps8)
    return z8[:N // 8].reshape(N, F_OUT)


# speedup vs baseline: 1.1288x; 1.0013x over previous
import functools

import jax
import jax.numpy as jnp
from jax import lax
from jax.experimental import pallas as pl
from jax.experimental.pallas import tpu as pltpu
from jax.experimental.pallas import tpu_sc as plsc

N = 10000
E = 320000
F_IN = 128
HID = 32
F_OUT = 16
NC, NS, L = 2, 16, 16
NW = NC * NS
CH = 128
NCH = E // CH
KB = 80
KT = NCH - KB * (NW - 1)
KTU = -(-KT // 8) * 8
N_ACC = 10240
RPT = N_ACC // NS
ZR = 40
NB = 6
NR4 = N_ACC // 4
XR4 = N // 4
NR8 = N_ACC // 8
MGRID, MBR = 8, N_ACC // 4 // 8
DGRID, DBR = 5, N_ACC // 8 // 5

_MESH = plsc.VectorSubcoreMesh(core_axis_name="c", subcore_axis_name="s")


@functools.partial(
    pl.kernel,
    out_type=jax.ShapeDtypeStruct((NC, N_ACC, HID), jnp.float32),
    mesh=_MESH,
    scratch_types=[
        pltpu.VMEM((KB * CH,), jnp.int32),
        pltpu.VMEM((CH, HID), jnp.float32),
        pltpu.VMEM((ZR, HID), jnp.float32),
        pltpu.VMEM_SHARED((N_ACC, HID), jnp.float32),
        pltpu.SemaphoreType.DMA,
    ],
    compiler_params=pltpu.CompilerParams(use_tc_tiling_on_sc=False),
)
def _deg_kernel(dstr, out, dst_v, ones_m, zbuf, acc, ssem):
    c = lax.axis_index("c")
    s = lax.axis_index("s")
    wid = s * NC + c
    start = KB * wid
    kw = jnp.where(wid < NW - 1, KB, KT)

    def _fill(i, _):
        ones_m[i // 2, pl.ds((i % 2) * L, L)] = jnp.full((L,), 1.0, jnp.float32)
        return 0

    lax.fori_loop(0, CH * (HID // L), _fill, 0)

    def _zb(i, _):
        zbuf[i // 2, pl.ds((i % 2) * L, L)] = jnp.zeros((L,), jnp.float32)
        return 0

    lax.fori_loop(0, ZR * (HID // L), _zb, 0)

    def _zc(j, _):
        pltpu.sync_copy(zbuf, acc.at[pl.ds(s * RPT + j * ZR, ZR)])
        return 0

    lax.fori_loop(0, RPT // ZR, _zc, 0)

    @pl.when(wid < NW - 1)
    def _():
        pltpu.sync_copy(dstr.at[pl.ds(start * CH, KB * CH)], dst_v)

    @pl.when(wid == NW - 1)
    def _():
        pltpu.sync_copy(dstr.at[pl.ds(start * CH, KTU * CH)], dst_v.at[pl.ds(0, KTU * CH)])

    plsc.subcore_barrier()

    def _scat(k, _):
        @pl.when(k >= 4)
        def _():
            pltpu.make_async_copy(ones_m, acc.at[dst_v.at[pl.ds((k - 4) * CH, CH)]], ssem).wait()

        pltpu.async_copy(ones_m, acc.at[dst_v.at[pl.ds(k * CH, CH)]], ssem, add=True)
        return 0

    lax.fori_loop(0, kw, _scat, 0)

    def _drain(k, _):
        pltpu.make_async_copy(ones_m, acc.at[dst_v.at[pl.ds(k * CH, CH)]], ssem).wait()
        return 0

    lax.fori_loop(kw - 4, kw, _drain, 0)
    plsc.subcore_barrier()
    pltpu.sync_copy(acc.at[pl.ds(s * RPT, RPT)], out.at[c, pl.ds(s * RPT, RPT)])


@functools.partial(
    pl.kernel,
    out_type=jax.ShapeDtypeStruct((NC, N_ACC, HID), jnp.float32),
    mesh=_MESH,
    scratch_types=[
        pltpu.VMEM((KB * CH,), jnp.int32),
        pltpu.VMEM((KB * CH,), jnp.int32),
        pltpu.VMEM((NB, CH, HID), jnp.float32),
        pltpu.VMEM((ZR, HID), jnp.float32),
        pltpu.VMEM_SHARED((N_ACC, HID), jnp.float32),
        pltpu.SemaphoreType.DMA,
        pltpu.SemaphoreType.DMA,
    ],
    compiler_params=pltpu.CompilerParams(use_tc_tiling_on_sc=False),
)
def _spmm_kernel(table, srcr, dstr, out, src_v, dst_v, rowbuf, zbuf, acc, gsem, ssem):
    c = lax.axis_index("c")
    s = lax.axis_index("s")
    wid = s * NC + c
    start = KB * wid
    kw = jnp.where(wid < NW - 1, KB, KT)

    def _zb(i, _):
        zbuf[i // 2, pl.ds((i % 2) * L, L)] = jnp.zeros((L,), jnp.float32)
        return 0

    lax.fori_loop(0, ZR * (HID // L), _zb, 0)

    def _zc(j, _):
        pltpu.sync_copy(zbuf, acc.at[pl.ds(s * RPT + j * ZR, ZR)])
        return 0

    lax.fori_loop(0, RPT // ZR, _zc, 0)
    @pl.when(wid < NW - 1)
    def _():
        pltpu.sync_copy(srcr.at[pl.ds(start * CH, KB * CH)], src_v)
        pltpu.sync_copy(dstr.at[pl.ds(start * CH, KB * CH)], dst_v)

    @pl.when(wid == NW - 1)
    def _():
        pltpu.sync_copy(srcr.at[pl.ds(start * CH, KTU * CH)], src_v.at[pl.ds(0, KTU * CH)])
        pltpu.sync_copy(dstr.at[pl.ds(start * CH, KTU * CH)], dst_v.at[pl.ds(0, KTU * CH)])

    plsc.subcore_barrier()

    for j in range(NB - 2):
        pltpu.async_copy(table.at[src_v.at[pl.ds(j * CH, CH)]], rowbuf.at[j], gsem)

    def _mb(k, _):
        b = lax.rem(k, NB)
        pb = lax.rem(k + NB - 2, NB)

        @pl.when(k >= 2)
        def _():
            pltpu.make_async_copy(rowbuf.at[pb], acc.at[dst_v.at[pl.ds((k - 2) * CH, CH)]], ssem).wait()

        @pl.when(k + NB - 2 < kw)
        def _():
            pltpu.async_copy(table.at[src_v.at[pl.ds((k + NB - 2) * CH, CH)]], rowbuf.at[pb], gsem)

        pltpu.make_async_copy(table.at[src_v.at[pl.ds(k * CH, CH)]], rowbuf.at[b], gsem).wait()
        pltpu.async_copy(rowbuf.at[b], acc.at[dst_v.at[pl.ds(k * CH, CH)]], ssem, add=True)
        return 0

    lax.fori_loop(0, kw, _mb, 0)
    pltpu.make_async_copy(rowbuf.at[lax.rem(kw - 2, NB)], acc.at[dst_v.at[pl.ds((kw - 2) * CH, CH)]], ssem).wait()
    pltpu.make_async_copy(rowbuf.at[lax.rem(kw - 1, NB)], acc.at[dst_v.at[pl.ds((kw - 1) * CH, CH)]], ssem).wait()
    plsc.subcore_barrier()
    pltpu.sync_copy(acc.at[pl.ds(s * RPT, RPT)], out.at[c, pl.ds(s * RPT, RPT)])


def _enc_body(degp_ref, x_ref, w1_ref, pt_ref, dinv_ref):
    deg4 = degp_ref[0] + degp_ref[1] + 1.0
    dinv4 = lax.rsqrt(jnp.maximum(deg4, 1.0))
    p = jnp.dot(x_ref[...], w1_ref[...], preferred_element_type=jnp.float32)
    pt_ref[pl.ds(0, XR4)] = p * dinv4[:XR4]
    dinv_ref[...] = dinv4


_enc = pl.pallas_call(
    _enc_body,
    out_shape=[
        jax.ShapeDtypeStruct((NR4, 128), jnp.float32),
        jax.ShapeDtypeStruct((NR4, 128), jnp.float32),
    ],
)


def _mid_body(y_ref, pt_ref, dinv_ref, b1_ref, ht_ref):
    t = y_ref[0] + y_ref[1] + pt_ref[...]
    dinv = dinv_ref[...]
    h = jnp.maximum(dinv * t + b1_ref[...], 0.0)
    ht_ref[...] = h * dinv


_mid = pl.pallas_call(
    _mid_body,
    grid=(MGRID,),
    in_specs=[
        pl.BlockSpec((2, MBR, 128), lambda i: (0, i, 0)),
        pl.BlockSpec((MBR, 128), lambda i: (i, 0)),
        pl.BlockSpec((MBR, 128), lambda i: (i, 0)),
        pl.BlockSpec((1, 128), lambda i: (0, 0)),
    ],
    out_specs=pl.BlockSpec((MBR, 128), lambda i: (i, 0)),
    out_shape=jax.ShapeDtypeStruct((NR4, 128), jnp.float32),
)


def _dec_body(z_ref, ht_ref, dinv_ref, wmu_ref, bmu_ref, wls_ref, bls_ref,
              eps_ref, z8_ref):
    agg = dinv_ref[...] * (z_ref[0] + z_ref[1] + ht_ref[...])
    mu = jnp.dot(agg, wmu_ref[...], preferred_element_type=jnp.float32) + bmu_ref[...]
    ls = jnp.dot(agg, wls_ref[...], preferred_element_type=jnp.float32) + bls_ref[...]
    z8_ref[...] = mu + eps_ref[...] * jnp.exp(ls)


_dec = pl.pallas_call(
    _dec_body,
    grid=(DGRID,),
    in_specs=[
        pl.BlockSpec((2, DBR, 256), lambda i: (0, i, 0)),
        pl.BlockSpec((DBR, 256), lambda i: (i, 0)),
        pl.BlockSpec((DBR, 256), lambda i: (i, 0)),
        pl.BlockSpec((256, 128), lambda i: (0, 0)),
        pl.BlockSpec((1, 128), lambda i: (0, 0)),
        pl.BlockSpec((256, 128), lambda i: (0, 0)),
        pl.BlockSpec((1, 128), lambda i: (0, 0)),
        pl.BlockSpec((DBR, 128), lambda i: (i, 0)),
    ],
    out_specs=pl.BlockSpec((DBR, 128), lambda i: (i, 0)),
    out_shape=jax.ShapeDtypeStruct((NR8, 128), jnp.float32),
)


def kernel(x, edge_index, W1, b1, Wmu, bmu, Wls, bls, eps):
    srcr = edge_index[0]
    dstr = edge_index[1]
    x4 = x.reshape(XR4, 4 * F_IN)
    f32 = jnp.float32
    w1b = jnp.kron(jnp.eye(4, dtype=f32), W1)
    wmub = jnp.kron(jnp.eye(8, dtype=f32), Wmu)
    wlsb = jnp.kron(jnp.eye(8, dtype=f32), Wls)
    b1b = jnp.tile(b1, 4)[None, :]
    bmub = jnp.tile(bmu, 8)[None, :]
    blsb = jnp.tile(bls, 8)[None, :]
    eps8 = jnp.pad(eps.reshape(N // 8, 128), ((0, NR8 - N // 8), (0, 0)))

    degp = _deg_kernel(dstr)
    pt4, dinv4 = _enc(degp.reshape(NC, NR4, 128), x4, w1b)
    y = _spmm_kernel(pt4.reshape(N_ACC, HID), srcr, dstr)
    ht4 = _mid(y.reshape(NC, NR4, 128), pt4, dinv4, b1b)
    z2 = _spmm_kernel(ht4.reshape(N_ACC, HID), srcr, dstr)
    z8 = _dec(z2.reshape(NC, NR8, 256), ht4.reshape(NR8, 256),
              dinv4.reshape(NR8, 256), wmub, bmub, wlsb, blsb, eps8)
    return z8[:N // 8].reshape(N, F_OUT)


# speedup vs baseline: 1.1294x; 1.0005x over previous
import functools

import jax
import jax.numpy as jnp
from jax import lax
from jax.experimental import pallas as pl
from jax.experimental.pallas import tpu as pltpu
from jax.experimental.pallas import tpu_sc as plsc

N = 10000
E = 320000
F_IN = 128
HID = 32
F_OUT = 16
NC, NS, L = 2, 16, 16
NW = NC * NS
CH = 128
NCH = E // CH
KB = 80
KT = NCH - KB * (NW - 1)
KTU = -(-KT // 8) * 8
N_ACC = 10240
RPT = N_ACC // NS
ZR = 40
NB = 6
NR4 = N_ACC // 4
XR4 = N // 4
NR8 = N_ACC // 8
MGRID, MBR = 8, N_ACC // 4 // 8
DGRID, DBR = 5, N_ACC // 8 // 5

_MESH = plsc.VectorSubcoreMesh(core_axis_name="c", subcore_axis_name="s")


@functools.partial(
    pl.kernel,
    out_type=jax.ShapeDtypeStruct((NC, N_ACC, HID), jnp.float32),
    mesh=_MESH,
    scratch_types=[
        pltpu.VMEM((KB * CH,), jnp.int32),
        pltpu.VMEM((CH, HID), jnp.float32),
        pltpu.VMEM((ZR, HID), jnp.float32),
        pltpu.VMEM_SHARED((N_ACC, HID), jnp.float32),
        pltpu.SemaphoreType.DMA,
    ],
    compiler_params=pltpu.CompilerParams(use_tc_tiling_on_sc=False),
)
def _deg_kernel(dstr, out, dst_v, ones_m, zbuf, acc, ssem):
    c = lax.axis_index("c")
    s = lax.axis_index("s")
    wid = s * NC + c
    start = KB * wid
    kw = jnp.where(wid < NW - 1, KB, KT)

    def _fill(i, _):
        ones_m[i // 2, pl.ds((i % 2) * L, L)] = jnp.full((L,), 1.0, jnp.float32)
        return 0

    lax.fori_loop(0, CH * (HID // L), _fill, 0)

    def _zb(i, _):
        zbuf[i // 2, pl.ds((i % 2) * L, L)] = jnp.zeros((L,), jnp.float32)
        return 0

    lax.fori_loop(0, ZR * (HID // L), _zb, 0)

    def _zc(j, _):
        pltpu.sync_copy(zbuf, acc.at[pl.ds(s * RPT + j * ZR, ZR)])
        return 0

    lax.fori_loop(0, RPT // ZR, _zc, 0)

    @pl.when(wid < NW - 1)
    def _():
        pltpu.sync_copy(dstr.at[pl.ds(start * CH, KB * CH)], dst_v)

    @pl.when(wid == NW - 1)
    def _():
        pltpu.sync_copy(dstr.at[pl.ds(start * CH, KTU * CH)], dst_v.at[pl.ds(0, KTU * CH)])

    plsc.subcore_barrier()

    def _scat(k, _):
        @pl.when(k >= 4)
        def _():
            pltpu.make_async_copy(ones_m, acc.at[dst_v.at[pl.ds((k - 4) * CH, CH)]], ssem).wait()

        pltpu.async_copy(ones_m, acc.at[dst_v.at[pl.ds(k * CH, CH)]], ssem, add=True)
        return 0

    lax.fori_loop(0, kw, _scat, 0)

    def _drain(k, _):
        pltpu.make_async_copy(ones_m, acc.at[dst_v.at[pl.ds(k * CH, CH)]], ssem).wait()
        return 0

    lax.fori_loop(kw - 4, kw, _drain, 0)
    plsc.subcore_barrier()
    pltpu.sync_copy(acc.at[pl.ds(s * RPT, RPT)], out.at[c, pl.ds(s * RPT, RPT)])


@functools.partial(
    pl.kernel,
    out_type=jax.ShapeDtypeStruct((NC, N_ACC, HID), jnp.float32),
    mesh=_MESH,
    scratch_types=[
        pltpu.VMEM((KB * CH,), jnp.int32),
        pltpu.VMEM((KB * CH,), jnp.int32),
        pltpu.VMEM((NB, CH, HID), jnp.float32),
        pltpu.VMEM((ZR, HID), jnp.float32),
        pltpu.VMEM_SHARED((N_ACC, HID), jnp.float32),
        pltpu.SemaphoreType.DMA,
        pltpu.SemaphoreType.DMA,
    ],
    compiler_params=pltpu.CompilerParams(use_tc_tiling_on_sc=False),
)
def _spmm_kernel(table, srcr, dstr, out, src_v, dst_v, rowbuf, zbuf, acc, gsem, ssem):
    c = lax.axis_index("c")
    s = lax.axis_index("s")
    wid = s * NC + c
    start = KB * wid
    kw = jnp.where(wid < NW - 1, KB, KT)

    def _zb(i, _):
        zbuf[i // 2, pl.ds((i % 2) * L, L)] = jnp.zeros((L,), jnp.float32)
        return 0

    lax.fori_loop(0, ZR * (HID // L), _zb, 0)

    def _zc(j, _):
        pltpu.sync_copy(zbuf, acc.at[pl.ds(s * RPT + j * ZR, ZR)])
        return 0

    lax.fori_loop(0, RPT // ZR, _zc, 0)
    @pl.when(wid < NW - 1)
    def _():
        pltpu.sync_copy(srcr.at[pl.ds(start * CH, KB * CH)], src_v)
        pltpu.sync_copy(dstr.at[pl.ds(start * CH, KB * CH)], dst_v)

    @pl.when(wid == NW - 1)
    def _():
        pltpu.sync_copy(srcr.at[pl.ds(start * CH, KTU * CH)], src_v.at[pl.ds(0, KTU * CH)])
        pltpu.sync_copy(dstr.at[pl.ds(start * CH, KTU * CH)], dst_v.at[pl.ds(0, KTU * CH)])

    plsc.subcore_barrier()

    for j in range(NB - 2):
        pltpu.async_copy(table.at[src_v.at[pl.ds(j * CH, CH)]], rowbuf.at[j], gsem)

    def _mb(k, _):
        b = lax.rem(k, NB)
        pb = lax.rem(k + NB - 2, NB)

        @pl.when(k >= 2)
        def _():
            pltpu.make_async_copy(rowbuf.at[pb], acc.at[dst_v.at[pl.ds((k - 2) * CH, CH)]], ssem).wait()

        @pl.when(k + NB - 2 < kw)
        def _():
            pltpu.async_copy(table.at[src_v.at[pl.ds((k + NB - 2) * CH, CH)]], rowbuf.at[pb], gsem)

        pltpu.make_async_copy(table.at[src_v.at[pl.ds(k * CH, CH)]], rowbuf.at[b], gsem).wait()
        pltpu.async_copy(rowbuf.at[b], acc.at[dst_v.at[pl.ds(k * CH, CH)]], ssem, add=True)
        return 0

    lax.fori_loop(0, kw, _mb, 0)
    pltpu.make_async_copy(rowbuf.at[lax.rem(kw - 2, NB)], acc.at[dst_v.at[pl.ds((kw - 2) * CH, CH)]], ssem).wait()
    pltpu.make_async_copy(rowbuf.at[lax.rem(kw - 1, NB)], acc.at[dst_v.at[pl.ds((kw - 1) * CH, CH)]], ssem).wait()
    plsc.subcore_barrier()
    pltpu.sync_copy(acc.at[pl.ds(s * RPT, RPT)], out.at[c, pl.ds(s * RPT, RPT)])


def _mm_body(x_ref, w1_ref, p_ref):
    p_ref[...] = jnp.dot(x_ref[...], w1_ref[...],
                         preferred_element_type=jnp.float32)


_mm = pl.pallas_call(
    _mm_body,
    out_shape=jax.ShapeDtypeStruct((XR4, 128), jnp.float32),
)


def _enc_body(degp_ref, p_ref, pt_ref, dinv_ref):
    deg4 = degp_ref[0] + degp_ref[1] + 1.0
    dinv4 = lax.rsqrt(jnp.maximum(deg4, 1.0))
    pt_ref[pl.ds(0, XR4)] = p_ref[...] * dinv4[:XR4]
    dinv_ref[...] = dinv4


_enc = pl.pallas_call(
    _enc_body,
    out_shape=[
        jax.ShapeDtypeStruct((NR4, 128), jnp.float32),
        jax.ShapeDtypeStruct((NR4, 128), jnp.float32),
    ],
)


def _mid_body(y_ref, pt_ref, dinv_ref, b1_ref, ht_ref):
    t = y_ref[0] + y_ref[1] + pt_ref[...]
    dinv = dinv_ref[...]
    h = jnp.maximum(dinv * t + b1_ref[...], 0.0)
    ht_ref[...] = h * dinv


_mid = pl.pallas_call(
    _mid_body,
    grid=(MGRID,),
    in_specs=[
        pl.BlockSpec((2, MBR, 128), lambda i: (0, i, 0)),
        pl.BlockSpec((MBR, 128), lambda i: (i, 0)),
        pl.BlockSpec((MBR, 128), lambda i: (i, 0)),
        pl.BlockSpec((1, 128), lambda i: (0, 0)),
    ],
    out_specs=pl.BlockSpec((MBR, 128), lambda i: (i, 0)),
    out_shape=jax.ShapeDtypeStruct((NR4, 128), jnp.float32),
)


def _dec_body(z_ref, ht_ref, dinv_ref, wmu_ref, bmu_ref, wls_ref, bls_ref,
              eps_ref, z8_ref):
    agg = dinv_ref[...] * (z_ref[0] + z_ref[1] + ht_ref[...])
    mu = jnp.dot(agg, wmu_ref[...], preferred_element_type=jnp.float32) + bmu_ref[...]
    ls = jnp.dot(agg, wls_ref[...], preferred_element_type=jnp.float32) + bls_ref[...]
    z8_ref[...] = mu + eps_ref[...] * jnp.exp(ls)


_dec = pl.pallas_call(
    _dec_body,
    grid=(DGRID,),
    in_specs=[
        pl.BlockSpec((2, DBR, 256), lambda i: (0, i, 0)),
        pl.BlockSpec((DBR, 256), lambda i: (i, 0)),
        pl.BlockSpec((DBR, 256), lambda i: (i, 0)),
        pl.BlockSpec((256, 128), lambda i: (0, 0)),
        pl.BlockSpec((1, 128), lambda i: (0, 0)),
        pl.BlockSpec((256, 128), lambda i: (0, 0)),
        pl.BlockSpec((1, 128), lambda i: (0, 0)),
        pl.BlockSpec((DBR, 128), lambda i: (i, 0)),
    ],
    out_specs=pl.BlockSpec((DBR, 128), lambda i: (i, 0)),
    out_shape=jax.ShapeDtypeStruct((NR8, 128), jnp.float32),
)


def kernel(x, edge_index, W1, b1, Wmu, bmu, Wls, bls, eps):
    srcr = edge_index[0]
    dstr = edge_index[1]
    x4 = x.reshape(XR4, 4 * F_IN)
    f32 = jnp.float32
    w1b = jnp.kron(jnp.eye(4, dtype=f32), W1)
    wmub = jnp.kron(jnp.eye(8, dtype=f32), Wmu)
    wlsb = jnp.kron(jnp.eye(8, dtype=f32), Wls)
    b1b = jnp.tile(b1, 4)[None, :]
    bmub = jnp.tile(bmu, 8)[None, :]
    blsb = jnp.tile(bls, 8)[None, :]
    eps8 = jnp.pad(eps.reshape(N // 8, 128), ((0, NR8 - N // 8), (0, 0)))

    p4 = _mm(x4, w1b)
    degp = _deg_kernel(dstr)
    pt4, dinv4 = _enc(degp.reshape(NC, NR4, 128), p4)
    y = _spmm_kernel(pt4.reshape(N_ACC, HID), srcr, dstr)
    ht4 = _mid(y.reshape(NC, NR4, 128), pt4, dinv4, b1b)
    z2 = _spmm_kernel(ht4.reshape(N_ACC, HID), srcr, dstr)
    z8 = _dec(z2.reshape(NC, NR8, 256), ht4.reshape(NR8, 256),
              dinv4.reshape(NR8, 256), wmub, bmub, wlsb, blsb, eps8)
    return z8[:N // 8].reshape(N, F_OUT)
